# baseline (speedup 1.0000x reference)
import functools
import math

import jax
import jax.numpy as jnp
import numpy as np
from jax import lax
from jax.experimental import pallas as pl
from jax.experimental.pallas import tpu as pltpu

_LEVELS = [15, 15, 15]
_C = len(_LEVELS)
_CP = 128
_EPS = 1e-3

_HALF_L = (_LEVELS[0] - 1.0) * (1.0 + _EPS) / 2.0
_OFFSET = 0.5 if _LEVELS[0] % 2 == 0 else 0.0
_SHIFT = math.atanh(_OFFSET / _HALF_L)
_HALF_WIDTH = float(np.floor(_LEVELS[0] / 2.0))

_CH_R = 512
_DEPTH = 8


def _fsq_kernel(z_hbm, wc_ref, bc_ref, we_ref, be_ref, traw_ref,
                zq_hbm, err_ref, inbuf, outbuf, sin, sout, *, n_valid, nch):
    def in_copy(ch, slot):
        return pltpu.make_async_copy(
            z_hbm.at[pl.ds(ch * _CH_R, _CH_R), :], inbuf.at[slot],
            sin.at[slot])

    def out_copy(ch, slot):
        return pltpu.make_async_copy(
            outbuf.at[slot], zq_hbm.at[pl.ds(ch * _CH_R, _CH_R), :],
            sout.at[slot])

    for d in range(_DEPTH):
        in_copy(d, d).start()

    t = jax.nn.softplus(traw_ref[...])
    inv_t = 1.0 / t
    scale = t * (1.0 / _HALF_WIDTH)
    wc = wc_ref[...]
    bc = bc_ref[...]
    we = we_ref[...]
    be = be_ref[...]

    def step(i, err_acc):
        slot = lax.rem(i, _DEPTH)
        in_copy(i, slot).wait()

        zc = lax.dot_general(
            inbuf[slot], wc, (((1,), (0,)), ((), ())),
            preferred_element_type=jnp.float32,
            precision=lax.Precision.DEFAULT,
        ) + bc

        zb = jnp.tanh(zc * inv_t + _SHIFT) * _HALF_L - _OFFSET
        codes = jnp.round(zb) * scale

        d_ = zc - codes
        err_acc = err_acc + jnp.sum(d_ * d_)

        @pl.when(i >= _DEPTH)
        def _():
            out_copy(i - _DEPTH, slot).wait()

        acc = codes[:, 0:1] * we[0:1, :]
        acc = acc + codes[:, 1:2] * we[1:2, :]
        acc = acc + codes[:, 2:3] * we[2:3, :]
        outbuf[slot] = acc + be

        out_copy(i, slot).start()

        @pl.when(i + _DEPTH < nch)
        def _():
            in_copy(i + _DEPTH, slot).start()

        return err_acc

    err_acc = lax.fori_loop(0, nch, step, jnp.float32(0.0))
    err_ref[...] = (err_acc * (1.0 / n_valid)).reshape(1, 1)

    for d in range(_DEPTH):
        ch = nch - _DEPTH + d
        out_copy(ch, ch % _DEPTH).wait()


def kernel(z, W_c, b_c, W_e, b_e, T_raw):
    B, S, D = z.shape
    N = B * S
    z2 = z.reshape(N, D)
    nch = N // _CH_R

    wc_t = jnp.zeros((D, _CP), jnp.float32).at[:, :_C].set(W_c.T)
    bc = jnp.zeros((1, _CP), jnp.float32).at[0, :_C].set(b_c)
    we_t = jnp.zeros((8, D), jnp.float32).at[:_C, :].set(W_e.T)
    be = b_e.reshape(1, D).astype(jnp.float32)
    traw = jnp.zeros((1, _CP), jnp.float32).at[0, :_C].set(T_raw)

    vmem = pltpu.MemorySpace.VMEM
    hbm = pltpu.MemorySpace.HBM
    zq, err = pl.pallas_call(
        functools.partial(_fsq_kernel, n_valid=float(N * _C), nch=nch),
        in_specs=[
            pl.BlockSpec(memory_space=hbm),
            pl.BlockSpec(memory_space=vmem),
            pl.BlockSpec(memory_space=vmem),
            pl.BlockSpec(memory_space=vmem),
            pl.BlockSpec(memory_space=vmem),
            pl.BlockSpec(memory_space=vmem),
        ],
        out_specs=[
            pl.BlockSpec(memory_space=hbm),
            pl.BlockSpec(memory_space=vmem),
        ],
        out_shape=[
            jax.ShapeDtypeStruct((N, D), jnp.float32),
            jax.ShapeDtypeStruct((1, 1), jnp.float32),
        ],
        scratch_shapes=[
            pltpu.VMEM((_DEPTH, _CH_R, D), jnp.float32),
            pltpu.VMEM((_DEPTH, _CH_R, D), jnp.float32),
            pltpu.SemaphoreType.DMA((_DEPTH,)),
            pltpu.SemaphoreType.DMA((_DEPTH,)),
        ],
    )(z2, wc_t, bc, we_t, be, traw)

    return zq.reshape(B, S, D), err[0, 0]


# speedup vs baseline: 1.1265x; 1.1265x over previous
import functools
import math

import jax
import jax.numpy as jnp
import numpy as np
from jax import lax
from jax.experimental import pallas as pl
from jax.experimental.pallas import tpu as pltpu

_LEVELS = [15, 15, 15]
_C = len(_LEVELS)
_CP = 128
_EPS = 1e-3

_HALF_L = (_LEVELS[0] - 1.0) * (1.0 + _EPS) / 2.0
_OFFSET = 0.5 if _LEVELS[0] % 2 == 0 else 0.0
_SHIFT = math.atanh(_OFFSET / _HALF_L)
_HALF_WIDTH = float(np.floor(_LEVELS[0] / 2.0))

_CH_R = 2048
_DEPTH = 3


def _fsq_kernel(z_hbm, wc_ref, bc_ref, we_ref, be_ref, traw_ref,
                zq_hbm, err_ref, inbuf, outbuf, sin, sout, *, n_valid, nch):
    def in_copy(ch, slot):
        return pltpu.make_async_copy(
            z_hbm.at[pl.ds(ch * _CH_R, _CH_R), :], inbuf.at[slot],
            sin.at[slot])

    def out_copy(ch, slot):
        return pltpu.make_async_copy(
            outbuf.at[slot], zq_hbm.at[pl.ds(ch * _CH_R, _CH_R), :],
            sout.at[slot])

    for d in range(_DEPTH):
        in_copy(d, d).start()

    t = jax.nn.softplus(traw_ref[...])
    inv_t = 1.0 / t
    scale = t * (1.0 / _HALF_WIDTH)
    wc = wc_ref[...]
    bc = bc_ref[...]
    we = we_ref[...]
    be = be_ref[...]

    def step(i, err_acc):
        slot = lax.rem(i, _DEPTH)
        in_copy(i, slot).wait()

        zc = lax.dot_general(
            inbuf[slot], wc, (((1,), (0,)), ((), ())),
            preferred_element_type=jnp.float32,
            precision=lax.Precision.DEFAULT,
        ) + bc

        zb = jnp.tanh(zc * inv_t + _SHIFT) * _HALF_L - _OFFSET
        codes = jnp.round(zb) * scale

        d_ = zc - codes
        err_acc = err_acc + jnp.sum(d_ * d_)

        @pl.when(i >= _DEPTH)
        def _():
            out_copy(i - _DEPTH, slot).wait()

        acc = codes[:, 0:1] * we[0:1, :]
        acc = acc + codes[:, 1:2] * we[1:2, :]
        acc = acc + codes[:, 2:3] * we[2:3, :]
        outbuf[slot] = acc + be

        out_copy(i, slot).start()

        @pl.when(i + _DEPTH < nch)
        def _():
            in_copy(i + _DEPTH, slot).start()

        return err_acc

    err_acc = lax.fori_loop(0, nch, step, jnp.float32(0.0))
    err_ref[...] = (err_acc * (1.0 / n_valid)).reshape(1, 1)

    for d in range(_DEPTH):
        ch = nch - _DEPTH + d
        out_copy(ch, ch % _DEPTH).wait()


def kernel(z, W_c, b_c, W_e, b_e, T_raw):
    B, S, D = z.shape
    N = B * S
    z2 = z.reshape(N, D)
    nch = N // _CH_R

    wc_t = jnp.zeros((D, _CP), jnp.float32).at[:, :_C].set(W_c.T)
    bc = jnp.zeros((1, _CP), jnp.float32).at[0, :_C].set(b_c)
    we_t = jnp.zeros((8, D), jnp.float32).at[:_C, :].set(W_e.T)
    be = b_e.reshape(1, D).astype(jnp.float32)
    traw = jnp.zeros((1, _CP), jnp.float32).at[0, :_C].set(T_raw)

    vmem = pltpu.MemorySpace.VMEM
    hbm = pltpu.MemorySpace.HBM
    zq, err = pl.pallas_call(
        functools.partial(_fsq_kernel, n_valid=float(N * _C), nch=nch),
        in_specs=[
            pl.BlockSpec(memory_space=hbm),
            pl.BlockSpec(memory_space=vmem),
            pl.BlockSpec(memory_space=vmem),
            pl.BlockSpec(memory_space=vmem),
            pl.BlockSpec(memory_space=vmem),
            pl.BlockSpec(memory_space=vmem),
        ],
        out_specs=[
            pl.BlockSpec(memory_space=hbm),
            pl.BlockSpec(memory_space=vmem),
        ],
        out_shape=[
            jax.ShapeDtypeStruct((N, D), jnp.float32),
            jax.ShapeDtypeStruct((1, 1), jnp.float32),
        ],
        scratch_shapes=[
            pltpu.VMEM((_DEPTH, _CH_R, D), jnp.float32),
            pltpu.VMEM((_DEPTH, _CH_R, D), jnp.float32),
            pltpu.SemaphoreType.DMA((_DEPTH,)),
            pltpu.SemaphoreType.DMA((_DEPTH,)),
        ],
    )(z2, wc_t, bc, we_t, be, traw)

    return zq.reshape(B, S, D), err[0, 0]


# speedup vs baseline: 1.1298x; 1.0029x over previous
import functools
import math

import jax
import jax.numpy as jnp
import numpy as np
from jax import lax
from jax.experimental import pallas as pl
from jax.experimental.pallas import tpu as pltpu

_LEVELS = [15, 15, 15]
_C = len(_LEVELS)
_CP = 128
_EPS = 1e-3

_HALF_L = (_LEVELS[0] - 1.0) * (1.0 + _EPS) / 2.0
_OFFSET = 0.5 if _LEVELS[0] % 2 == 0 else 0.0
_SHIFT = math.atanh(_OFFSET / _HALF_L)
_HALF_WIDTH = float(np.floor(_LEVELS[0] / 2.0))

_CH_R = 2048
_DEPTH = 4


def _fsq_kernel(z_hbm, wc_ref, bc_ref, we_ref, be_ref, traw_ref,
                zq_hbm, err_ref, inbuf, outbuf, sin, sout, *, n_valid, nch):
    def in_copy(ch, slot):
        return pltpu.make_async_copy(
            z_hbm.at[pl.ds(ch * _CH_R, _CH_R), :], inbuf.at[slot],
            sin.at[slot])

    def out_copy(ch, slot):
        return pltpu.make_async_copy(
            outbuf.at[slot], zq_hbm.at[pl.ds(ch * _CH_R, _CH_R), :],
            sout.at[slot])

    for d in range(_DEPTH):
        in_copy(d, d).start()

    t = jax.nn.softplus(traw_ref[...])
    inv_t = 1.0 / t
    scale = t * (1.0 / _HALF_WIDTH)
    wc = wc_ref[...]
    bc = bc_ref[...]
    we = we_ref[...]
    be = be_ref[...]

    def step(i, err_acc):
        slot = lax.rem(i, _DEPTH)
        in_copy(i, slot).wait()

        zc = lax.dot_general(
            inbuf[slot], wc, (((1,), (0,)), ((), ())),
            preferred_element_type=jnp.float32,
            precision=lax.Precision.DEFAULT,
        ) + bc

        zb = jnp.tanh(zc * inv_t + _SHIFT) * _HALF_L - _OFFSET
        codes = jnp.round(zb) * scale

        d_ = zc - codes
        err_acc = err_acc + jnp.sum(d_ * d_)

        @pl.when(i >= _DEPTH)
        def _():
            out_copy(i - _DEPTH, slot).wait()

        acc = codes[:, 0:1] * we[0:1, :]
        acc = acc + codes[:, 1:2] * we[1:2, :]
        acc = acc + codes[:, 2:3] * we[2:3, :]
        outbuf[slot] = acc + be

        out_copy(i, slot).start()

        @pl.when(i + _DEPTH < nch)
        def _():
            in_copy(i + _DEPTH, slot).start()

        return err_acc

    err_acc = lax.fori_loop(0, nch, step, jnp.float32(0.0))
    err_ref[...] = (err_acc * (1.0 / n_valid)).reshape(1, 1)

    for d in range(_DEPTH):
        ch = nch - _DEPTH + d
        out_copy(ch, ch % _DEPTH).wait()


def kernel(z, W_c, b_c, W_e, b_e, T_raw):
    B, S, D = z.shape
    N = B * S
    z2 = z.reshape(N, D)
    nch = N // _CH_R

    wc_t = jnp.zeros((D, _CP), jnp.float32).at[:, :_C].set(W_c.T)
    bc = jnp.zeros((1, _CP), jnp.float32).at[0, :_C].set(b_c)
    we_t = jnp.zeros((8, D), jnp.float32).at[:_C, :].set(W_e.T)
    be = b_e.reshape(1, D).astype(jnp.float32)
    traw = jnp.zeros((1, _CP), jnp.float32).at[0, :_C].set(T_raw)

    vmem = pltpu.MemorySpace.VMEM
    hbm = pltpu.MemorySpace.HBM
    zq, err = pl.pallas_call(
        functools.partial(_fsq_kernel, n_valid=float(N * _C), nch=nch),
        in_specs=[
            pl.BlockSpec(memory_space=hbm),
            pl.BlockSpec(memory_space=vmem),
            pl.BlockSpec(memory_space=vmem),
            pl.BlockSpec(memory_space=vmem),
            pl.BlockSpec(memory_space=vmem),
            pl.BlockSpec(memory_space=vmem),
        ],
        out_specs=[
            pl.BlockSpec(memory_space=hbm),
            pl.BlockSpec(memory_space=vmem),
        ],
        out_shape=[
            jax.ShapeDtypeStruct((N, D), jnp.float32),
            jax.ShapeDtypeStruct((1, 1), jnp.float32),
        ],
        scratch_shapes=[
            pltpu.VMEM((_DEPTH, _CH_R, D), jnp.float32),
            pltpu.VMEM((_DEPTH, _CH_R, D), jnp.float32),
            pltpu.SemaphoreType.DMA((_DEPTH,)),
            pltpu.SemaphoreType.DMA((_DEPTH,)),
        ],
    )(z2, wc_t, bc, we_t, be, traw)

    return zq.reshape(B, S, D), err[0, 0]
